# baseline (speedup 1.0000x reference)
import functools

import jax
import jax.numpy as jnp
from jax import lax
from jax.experimental import pallas as pl
from jax.experimental.pallas import tpu as pltpu
from jax.experimental.pallas import tpu_sc as plsc

MAX_LEN = 2048
S = 2048
D = 64
T = 2 * MAX_LEN - 1
NC, NS = 2, 16
NW = NC * NS
ROWS_PER = S // NW

_mesh = plsc.VectorSubcoreMesh(
    core_axis_name="c", subcore_axis_name="s", num_cores=NC, num_subcores=NS)


@functools.partial(
    pl.kernel,
    out_type=jax.ShapeDtypeStruct((S, S, D), jnp.float32),
    mesh=_mesh,
    scratch_types=[
        pltpu.VMEM_SHARED((T, D), jnp.float32),
        pltpu.SemaphoreType.DMA,
    ],
)
def _sc_toeplitz(table_hbm, out_hbm, table_sp, sem):
    s = lax.axis_index("s")
    c = lax.axis_index("c")

    @pl.when(s == 0)
    def _stage():
        pltpu.async_copy(table_hbm, table_sp, sem).wait()

    plsc.subcore_barrier()

    wid = s * NC + c
    base = wid * ROWS_PER

    def body(r, carry):
        i = base + r
        start = (MAX_LEN - 1) - i
        pltpu.async_copy(table_sp.at[pl.ds(start, S)], out_hbm.at[i], sem).wait()
        return carry

    lax.fori_loop(0, ROWS_PER, body, 0)


def kernel(seq_len, relative_positions):
    del seq_len
    return _sc_toeplitz(relative_positions)


# speedup vs baseline: 1.0080x; 1.0080x over previous
import functools

import jax
import jax.numpy as jnp
from jax import lax
from jax.experimental import pallas as pl
from jax.experimental.pallas import tpu as pltpu
from jax.experimental.pallas import tpu_sc as plsc

MAX_LEN = 2048
S = 2048
D = 64
T = 2 * MAX_LEN - 1
NC, NS = 2, 16
NW = NC * NS
ROWS_PER = S // NW

_mesh = plsc.VectorSubcoreMesh(
    core_axis_name="c", subcore_axis_name="s", num_cores=NC, num_subcores=NS)


@functools.partial(
    pl.kernel,
    out_type=jax.ShapeDtypeStruct((S, S, D), jnp.float32),
    mesh=_mesh,
    scratch_types=[
        pltpu.VMEM_SHARED((T, D), jnp.float32),
        pltpu.SemaphoreType.DMA,
    ],
)
def _sc_toeplitz(table_hbm, out_hbm, table_sp, sem):
    s = lax.axis_index("s")
    c = lax.axis_index("c")

    @pl.when(s == 0)
    def _stage():
        pltpu.async_copy(table_hbm, table_sp, sem).wait()

    plsc.subcore_barrier()

    wid = s * NC + c
    base = wid * ROWS_PER

    def _desc(i):
        start = (MAX_LEN - 1) - i
        return pltpu.make_async_copy(
            table_sp.at[pl.ds(start, S)], out_hbm.at[i], sem)

    PIPE = 8
    for r in range(PIPE):
        _desc(base + r).start()

    def body(r, carry):
        _desc(base + r + PIPE).start()
        _desc(base + r).wait()
        return carry

    lax.fori_loop(0, ROWS_PER - PIPE, body, 0)
    for r in range(PIPE):
        _desc(base + ROWS_PER - PIPE + r).wait()


def kernel(seq_len, relative_positions):
    del seq_len
    return _sc_toeplitz(relative_positions)


# speedup vs baseline: 2.1741x; 2.1569x over previous
import jax
import jax.numpy as jnp
from jax.experimental import pallas as pl

MAX_LEN = 2048
S = 2048
D = 64
T = 2 * MAX_LEN - 1
DT = D // 8
JT = S // 128


def _body(table_ref, out_ref):
    i = pl.program_id(0)
    m0 = (MAX_LEN - 1) - i
    for jt in range(JT):
        src = table_ref[pl.ds(m0 + 128 * jt, 128), :]
        out_ref[0, :, jt] = jnp.transpose(src, (1, 0)).reshape(DT, 8, 128)


def kernel(seq_len, relative_positions):
    del seq_len
    res5 = pl.pallas_call(
        _body,
        grid=(S,),
        in_specs=[pl.BlockSpec((T, D), lambda i: (0, 0))],
        out_specs=pl.BlockSpec((1, DT, JT, 8, 128), lambda i: (i, 0, 0, 0, 0)),
        out_shape=jax.ShapeDtypeStruct((S, DT, JT, 8, 128), jnp.float32),
    )(relative_positions)
    return res5.transpose(0, 2, 4, 1, 3).reshape(S, S, D)


# speedup vs baseline: 3.1097x; 1.4303x over previous
import jax
import jax.numpy as jnp
from jax.experimental import pallas as pl

MAX_LEN = 2048
S = 2048
D = 64
T = 2 * MAX_LEN - 1
DT = D // 8
JT = S // 128
BI = 2


def _body(table_ref, out_ref):
    b = pl.program_id(0)
    for r in range(BI):
        m0 = (MAX_LEN - 1) - (b * BI + r)
        for jt in range(JT):
            src = table_ref[pl.ds(m0 + 128 * jt, 128), :]
            out_ref[r, :, jt] = jnp.transpose(src, (1, 0)).reshape(DT, 8, 128)


def kernel(seq_len, relative_positions):
    del seq_len
    res5 = pl.pallas_call(
        _body,
        grid=(S // BI,),
        in_specs=[pl.BlockSpec((T, D), lambda b: (0, 0))],
        out_specs=pl.BlockSpec((BI, DT, JT, 8, 128), lambda b: (b, 0, 0, 0, 0)),
        out_shape=jax.ShapeDtypeStruct((S, DT, JT, 8, 128), jnp.float32),
    )(relative_positions)
    return res5.transpose(0, 2, 4, 1, 3).reshape(S, S, D)


# speedup vs baseline: 4.4438x; 1.4290x over previous
import jax
import jax.numpy as jnp
from jax.experimental import pallas as pl

MAX_LEN = 2048
S = 2048
D = 64
T = 2 * MAX_LEN - 1
DT = D // 8
JT = S // 128
BI = 32


def _body(table_ref, out_ref):
    b = pl.program_id(0)
    for r in range(BI):
        m0 = (MAX_LEN - 1) - (b * BI + r)
        for jt in range(JT):
            src = table_ref[pl.ds(m0 + 128 * jt, 128), :]
            out_ref[r, :, jt] = jnp.transpose(src, (1, 0)).reshape(DT, 8, 128)


def kernel(seq_len, relative_positions):
    del seq_len
    res5 = pl.pallas_call(
        _body,
        grid=(S // BI,),
        in_specs=[pl.BlockSpec((T, D), lambda b: (0, 0))],
        out_specs=pl.BlockSpec((BI, DT, JT, 8, 128), lambda b: (b, 0, 0, 0, 0)),
        out_shape=jax.ShapeDtypeStruct((S, DT, JT, 8, 128), jnp.float32),
    )(relative_positions)
    return res5.transpose(0, 2, 4, 1, 3).reshape(S, S, D)
